# baseline (speedup 1.0000x reference)
import numpy as np
import jax
import jax.numpy as jnp
from jax.experimental import pallas as pl

_N = 50
_M = 56
_ROWS = 16384
_COLS = 100
_LANES = _COLS
_BM = 2048


def _build_consts():
    n = _N
    np.random.seed(40)
    xs = np.random.uniform(0, 3, n)
    np.random.seed(122)
    ys = np.random.uniform(0, 3, n)
    np.random.seed(36)
    noise = np.random.normal(0, 1, n)
    xs = np.asarray(xs, np.float32)
    ys = np.asarray(ys, np.float32)
    two = np.float32(2)
    four = np.float32(4)
    term1 = two * np.cos(two * xs) - (xs + ys) * four * np.sin(two * xs)
    term2 = two * np.cos(two * ys) - (xs + ys) * four * np.sin(two * ys)
    f_obs = (term1 + term2 + np.asarray(noise, np.float32)).astype(np.float32)
    u_x = (np.cos(two * xs) * two).astype(np.float32)
    u_y = (np.cos(two * ys) * two).astype(np.float32)

    crow = np.zeros((8, _LANES), np.float32)
    crow[0, :n] = xs
    crow[1, :n] = ys
    crow[2, :n] = u_x
    crow[3, :n] = u_y

    ccol = np.zeros((3 * _M, _LANES), np.float32)
    ccol[0:n, :] = xs[:, None]
    ccol[_M:_M + n, :] = ys[:, None]
    ccol[2 * _M:2 * _M + n, :] = f_obs[:, None]
    return jnp.asarray(crow), jnp.asarray(ccol)


_CROW, _CCOL = None, None


def _consts():
    global _CROW, _CCOL
    if _CROW is None:
        _CROW, _CCOL = _build_consts()
    return _CROW, _CCOL


def _nn_derivative_column(coord_col, coord_row, ku_row, jj, ii):
    diff = coord_col - coord_row
    dist = jnp.abs(diff)
    dist = jnp.where(jj == ii, 1e8, dist)
    dist = jnp.where(jj >= _N, 3e9, dist)
    min_d = jnp.min(dist, axis=1, keepdims=True)
    big_j = jnp.int32(2 ** 30)
    idx = jnp.min(jnp.where(dist == min_d, jj, big_j), axis=1, keepdims=True)
    onehot = (jj == idx).astype(jnp.float32)
    ku_nbr = jnp.sum(onehot * ku_row, axis=1, keepdims=True)
    d_nbr = jnp.sum(onehot * diff, axis=1, keepdims=True)
    ku_self = jnp.sum(jnp.where(jj == ii, ku_row, 0.0), axis=1, keepdims=True)
    return (ku_self - ku_nbr) / (d_nbr + 1e-8)


def _body(xq_ref, crow_ref, ccol_ref, x_ref, out_ref):
    jj = jax.lax.broadcasted_iota(jnp.int32, (_M, _LANES), 1)
    ii = jax.lax.broadcasted_iota(jnp.int32, (_M, _LANES), 0)

    q_row = xq_ref[0:1, :]
    q_row = jnp.clip(q_row, -10.0, 10.0)
    xs_row = crow_ref[0:1, :]
    ys_row = crow_ref[1:2, :]
    u_x_row = crow_ref[2:3, :]
    u_y_row = crow_ref[3:4, :]
    ku_x_row = jnp.clip(q_row * u_x_row, -1e6, 1e6)
    ku_y_row = jnp.clip(q_row * u_y_row, -1e6, 1e6)

    xs_col = ccol_ref[0:_M, :]
    ys_col = ccol_ref[_M:2 * _M, :]
    f_obs_col = ccol_ref[2 * _M:3 * _M, 0:1]

    d_ku_dx = _nn_derivative_column(xs_col, xs_row, ku_x_row, jj, ii)
    d_ku_dy = _nn_derivative_column(ys_col, ys_row, ku_y_row, jj, ii)
    f_hat = jnp.clip(d_ku_dx + d_ku_dy, -200.0, 200.0)
    diff = f_obs_col - f_hat
    ii_col = jax.lax.broadcasted_iota(jnp.int32, (_M, 1), 0)
    diff = jnp.where(ii_col < _N, diff, 0.0)
    u_val = 0.5 * jnp.sum(diff * diff)

    xb = x_ref[...]
    cc = jax.lax.broadcasted_iota(jnp.int32, (_BM, _COLS), 1)
    sq = jnp.where(cc >= _N, xb * xb, 0.0)
    k_val = 0.5 * jnp.sum(sq, axis=1, keepdims=True)

    out_ref[...] = k_val + u_val


def kernel(x):
    crow, ccol = _consts()
    grid = (_ROWS // _BM,)
    return pl.pallas_call(
        _body,
        grid=grid,
        in_specs=[
            pl.BlockSpec((8, _LANES), lambda i: (0, 0)),
            pl.BlockSpec((8, _LANES), lambda i: (0, 0)),
            pl.BlockSpec((3 * _M, _LANES), lambda i: (0, 0)),
            pl.BlockSpec((_BM, _COLS), lambda i: (i, 0)),
        ],
        out_specs=pl.BlockSpec((_BM, 1), lambda i: (i, 0)),
        out_shape=jax.ShapeDtypeStruct((_ROWS, 1), jnp.float32),
    )(x, crow, ccol, x)


# speedup vs baseline: 1.0101x; 1.0101x over previous
import numpy as np
import jax
import jax.numpy as jnp
from jax.experimental import pallas as pl
from jax.experimental.pallas import tpu as pltpu

_N = 50
_M = 56
_ROWS = 16384
_COLS = 100
_LANES = _COLS
_BM = 2048


def _build_consts():
    n = _N
    np.random.seed(40)
    xs = np.random.uniform(0, 3, n)
    np.random.seed(122)
    ys = np.random.uniform(0, 3, n)
    np.random.seed(36)
    noise = np.random.normal(0, 1, n)
    xs = np.asarray(xs, np.float32)
    ys = np.asarray(ys, np.float32)
    two = np.float32(2)
    four = np.float32(4)
    term1 = two * np.cos(two * xs) - (xs + ys) * four * np.sin(two * xs)
    term2 = two * np.cos(two * ys) - (xs + ys) * four * np.sin(two * ys)
    f_obs = (term1 + term2 + np.asarray(noise, np.float32)).astype(np.float32)
    u_x = (np.cos(two * xs) * two).astype(np.float32)
    u_y = (np.cos(two * ys) * two).astype(np.float32)

    crow = np.zeros((8, _LANES), np.float32)
    crow[0, :n] = xs
    crow[1, :n] = ys
    crow[2, :n] = u_x
    crow[3, :n] = u_y

    ccol = np.zeros((3 * _M, _LANES), np.float32)
    ccol[0:n, :] = xs[:, None]
    ccol[_M:_M + n, :] = ys[:, None]
    ccol[2 * _M:2 * _M + n, :] = f_obs[:, None]
    return jnp.asarray(crow), jnp.asarray(ccol)


_CROW, _CCOL = None, None


def _consts():
    global _CROW, _CCOL
    if _CROW is None:
        _CROW, _CCOL = _build_consts()
    return _CROW, _CCOL


def _nn_derivative_column(coord_col, coord_row, ku_row, jj, ii):
    diff = coord_col - coord_row
    dist = jnp.abs(diff)
    dist = jnp.where(jj == ii, 1e8, dist)
    dist = jnp.where(jj >= _N, 3e9, dist)
    min_d = jnp.min(dist, axis=1, keepdims=True)
    big_j = jnp.int32(2 ** 30)
    idx = jnp.min(jnp.where(dist == min_d, jj, big_j), axis=1, keepdims=True)
    onehot = (jj == idx).astype(jnp.float32)
    ku_nbr = jnp.sum(onehot * ku_row, axis=1, keepdims=True)
    d_nbr = jnp.sum(onehot * diff, axis=1, keepdims=True)
    ku_self = jnp.sum(jnp.where(jj == ii, ku_row, 0.0), axis=1, keepdims=True)
    return (ku_self - ku_nbr) / (d_nbr + 1e-8)


def _body(xq_ref, crow_ref, ccol_ref, x_ref, out_ref, u_scr):
    @pl.when(pl.program_id(0) == 0)
    def _():
        jj = jax.lax.broadcasted_iota(jnp.int32, (_M, _LANES), 1)
        ii = jax.lax.broadcasted_iota(jnp.int32, (_M, _LANES), 0)

        q_row = xq_ref[0:1, :]
        q_row = jnp.clip(q_row, -10.0, 10.0)
        xs_row = crow_ref[0:1, :]
        ys_row = crow_ref[1:2, :]
        u_x_row = crow_ref[2:3, :]
        u_y_row = crow_ref[3:4, :]
        ku_x_row = jnp.clip(q_row * u_x_row, -1e6, 1e6)
        ku_y_row = jnp.clip(q_row * u_y_row, -1e6, 1e6)

        xs_col = ccol_ref[0:_M, :]
        ys_col = ccol_ref[_M:2 * _M, :]
        f_obs_col = ccol_ref[2 * _M:3 * _M, 0:1]

        d_ku_dx = _nn_derivative_column(xs_col, xs_row, ku_x_row, jj, ii)
        d_ku_dy = _nn_derivative_column(ys_col, ys_row, ku_y_row, jj, ii)
        f_hat = jnp.clip(d_ku_dx + d_ku_dy, -200.0, 200.0)
        diff = f_obs_col - f_hat
        ii_col = jax.lax.broadcasted_iota(jnp.int32, (_M, 1), 0)
        diff = jnp.where(ii_col < _N, diff, 0.0)
        u_scr[0, 0] = 0.5 * jnp.sum(diff * diff)

    xb = x_ref[...]
    sq = xb * xb
    mrow = jax.lax.broadcasted_iota(jnp.int32, (_COLS, 1), 0)
    mvec = jnp.where(mrow >= _N, 0.5, 0.0).astype(jnp.float32)
    k_val = jax.lax.dot_general(
        sq, mvec,
        dimension_numbers=(((1,), (0,)), ((), ())),
        preferred_element_type=jnp.float32,
    )
    out_ref[...] = k_val + u_scr[0, 0]


def kernel(x):
    crow, ccol = _consts()
    grid = (_ROWS // _BM,)
    return pl.pallas_call(
        _body,
        grid=grid,
        in_specs=[
            pl.BlockSpec((8, _LANES), lambda i: (0, 0)),
            pl.BlockSpec((8, _LANES), lambda i: (0, 0)),
            pl.BlockSpec((3 * _M, _LANES), lambda i: (0, 0)),
            pl.BlockSpec((_BM, _COLS), lambda i: (i, 0)),
        ],
        out_specs=pl.BlockSpec((_BM, 1), lambda i: (i, 0)),
        out_shape=jax.ShapeDtypeStruct((_ROWS, 1), jnp.float32),
        scratch_shapes=[pltpu.SMEM((1, 1), jnp.float32)],
    )(x, crow, ccol, x)


# speedup vs baseline: 1.2030x; 1.1910x over previous
import numpy as np
import jax
import jax.numpy as jnp
from jax.experimental import pallas as pl
from jax.experimental.pallas import tpu as pltpu

_N = 50
_M = 56
_ROWS = 16384
_COLS = 100
_LANES = _COLS
_BM = 8192


def _build_consts():
    n = _N
    np.random.seed(40)
    xs = np.random.uniform(0, 3, n)
    np.random.seed(122)
    ys = np.random.uniform(0, 3, n)
    np.random.seed(36)
    noise = np.random.normal(0, 1, n)
    xs = np.asarray(xs, np.float32)
    ys = np.asarray(ys, np.float32)
    two = np.float32(2)
    four = np.float32(4)
    term1 = two * np.cos(two * xs) - (xs + ys) * four * np.sin(two * xs)
    term2 = two * np.cos(two * ys) - (xs + ys) * four * np.sin(two * ys)
    f_obs = (term1 + term2 + np.asarray(noise, np.float32)).astype(np.float32)
    u_x = (np.cos(two * xs) * two).astype(np.float32)
    u_y = (np.cos(two * ys) * two).astype(np.float32)

    crow = np.zeros((8, _LANES), np.float32)
    crow[0, :n] = xs
    crow[1, :n] = ys
    crow[2, :n] = u_x
    crow[3, :n] = u_y

    ccol = np.zeros((3 * _M, _LANES), np.float32)
    ccol[0:n, :] = xs[:, None]
    ccol[_M:_M + n, :] = ys[:, None]
    ccol[2 * _M:2 * _M + n, :] = f_obs[:, None]
    return jnp.asarray(crow), jnp.asarray(ccol)


_CROW, _CCOL = None, None


def _consts():
    global _CROW, _CCOL
    if _CROW is None:
        _CROW, _CCOL = _build_consts()
    return _CROW, _CCOL


def _nn_derivative_column(coord_col, coord_row, ku_row, jj, ii):
    diff = coord_col - coord_row
    dist = jnp.abs(diff)
    dist = jnp.where(jj == ii, 1e8, dist)
    dist = jnp.where(jj >= _N, 3e9, dist)
    min_d = jnp.min(dist, axis=1, keepdims=True)
    big_j = jnp.int32(2 ** 30)
    idx = jnp.min(jnp.where(dist == min_d, jj, big_j), axis=1, keepdims=True)
    onehot = (jj == idx).astype(jnp.float32)
    ku_nbr = jnp.sum(onehot * ku_row, axis=1, keepdims=True)
    d_nbr = jnp.sum(onehot * diff, axis=1, keepdims=True)
    ku_self = jnp.sum(jnp.where(jj == ii, ku_row, 0.0), axis=1, keepdims=True)
    return (ku_self - ku_nbr) / (d_nbr + 1e-8)


def _body(xq_ref, crow_ref, ccol_ref, x_ref, out_ref, u_scr):
    @pl.when(pl.program_id(0) == 0)
    def _():
        jj = jax.lax.broadcasted_iota(jnp.int32, (_M, _LANES), 1)
        ii = jax.lax.broadcasted_iota(jnp.int32, (_M, _LANES), 0)

        q_row = xq_ref[0:1, :]
        q_row = jnp.clip(q_row, -10.0, 10.0)
        xs_row = crow_ref[0:1, :]
        ys_row = crow_ref[1:2, :]
        u_x_row = crow_ref[2:3, :]
        u_y_row = crow_ref[3:4, :]
        ku_x_row = jnp.clip(q_row * u_x_row, -1e6, 1e6)
        ku_y_row = jnp.clip(q_row * u_y_row, -1e6, 1e6)

        xs_col = ccol_ref[0:_M, :]
        ys_col = ccol_ref[_M:2 * _M, :]
        f_obs_col = ccol_ref[2 * _M:3 * _M, 0:1]

        d_ku_dx = _nn_derivative_column(xs_col, xs_row, ku_x_row, jj, ii)
        d_ku_dy = _nn_derivative_column(ys_col, ys_row, ku_y_row, jj, ii)
        f_hat = jnp.clip(d_ku_dx + d_ku_dy, -200.0, 200.0)
        diff = f_obs_col - f_hat
        ii_col = jax.lax.broadcasted_iota(jnp.int32, (_M, 1), 0)
        diff = jnp.where(ii_col < _N, diff, 0.0)
        u_scr[0, 0] = 0.5 * jnp.sum(diff * diff)

    xb = x_ref[...]
    sq = xb * xb
    mrow = jax.lax.broadcasted_iota(jnp.int32, (_COLS, 1), 0)
    mvec = jnp.where(mrow >= _N, 0.5, 0.0).astype(jnp.float32)
    k_val = jax.lax.dot_general(
        sq, mvec,
        dimension_numbers=(((1,), (0,)), ((), ())),
        preferred_element_type=jnp.float32,
    )
    out_ref[...] = k_val + u_scr[0, 0]


def kernel(x):
    crow, ccol = _consts()
    grid = (_ROWS // _BM,)
    return pl.pallas_call(
        _body,
        grid=grid,
        in_specs=[
            pl.BlockSpec((8, _LANES), lambda i: (0, 0)),
            pl.BlockSpec((8, _LANES), lambda i: (0, 0)),
            pl.BlockSpec((3 * _M, _LANES), lambda i: (0, 0)),
            pl.BlockSpec((_BM, _COLS), lambda i: (i, 0)),
        ],
        out_specs=pl.BlockSpec((_BM, 1), lambda i: (i, 0)),
        out_shape=jax.ShapeDtypeStruct((_ROWS, 1), jnp.float32),
        scratch_shapes=[pltpu.SMEM((1, 1), jnp.float32)],
    )(x, crow, ccol, x)


# speedup vs baseline: 1.7167x; 1.4270x over previous
import numpy as np
import jax
import jax.numpy as jnp
from jax.experimental import pallas as pl
from jax.experimental.pallas import tpu as pltpu

_N = 50
_M = 56
_ROWS = 16384
_COLS = 100
_LANES = _COLS
_BM = 8192


def _build_consts():
    n = _N
    np.random.seed(40)
    xs = np.random.uniform(0, 3, n)
    np.random.seed(122)
    ys = np.random.uniform(0, 3, n)
    np.random.seed(36)
    noise = np.random.normal(0, 1, n)
    xs = np.asarray(xs, np.float32)
    ys = np.asarray(ys, np.float32)
    two = np.float32(2)
    four = np.float32(4)
    term1 = two * np.cos(two * xs) - (xs + ys) * four * np.sin(two * xs)
    term2 = two * np.cos(two * ys) - (xs + ys) * four * np.sin(two * ys)
    f_obs = (term1 + term2 + np.asarray(noise, np.float32)).astype(np.float32)
    u_x = (np.cos(two * xs) * two).astype(np.float32)
    u_y = (np.cos(two * ys) * two).astype(np.float32)

    crow = np.zeros((8, _LANES), np.float32)
    crow[0, :n] = xs
    crow[1, :n] = ys
    crow[2, :n] = u_x
    crow[3, :n] = u_y

    ccol = np.zeros((3 * _M, _LANES), np.float32)
    ccol[0:n, :] = xs[:, None]
    ccol[_M:_M + n, :] = ys[:, None]
    ccol[2 * _M:2 * _M + n, :] = f_obs[:, None]
    return jnp.asarray(crow), jnp.asarray(ccol)


_CROW, _CCOL = None, None


def _consts():
    global _CROW, _CCOL
    if _CROW is None:
        _CROW, _CCOL = _build_consts()
    return _CROW, _CCOL


def _nn_derivative_column(coord_col, coord_row, ku_row, jj, ii):
    diff = coord_col - coord_row
    dist = jnp.abs(diff)
    dist = jnp.where(jj == ii, 1e8, dist)
    dist = jnp.where(jj >= _N, 3e9, dist)
    min_d = jnp.min(dist, axis=1, keepdims=True)
    big_j = jnp.int32(2 ** 30)
    idx = jnp.min(jnp.where(dist == min_d, jj, big_j), axis=1, keepdims=True)
    onehot = (jj == idx).astype(jnp.float32)
    ku_nbr = jnp.sum(onehot * ku_row, axis=1, keepdims=True)
    d_nbr = jnp.sum(onehot * diff, axis=1, keepdims=True)
    ku_self = jnp.sum(jnp.where(jj == ii, ku_row, 0.0), axis=1, keepdims=True)
    return (ku_self - ku_nbr) / (d_nbr + 1e-8)


def _body(crow_ref, ccol_ref, x_ref, out_ref, u_scr):
    @pl.when(pl.program_id(0) == 0)
    def _():
        jj = jax.lax.broadcasted_iota(jnp.int32, (_M, _LANES), 1)
        ii = jax.lax.broadcasted_iota(jnp.int32, (_M, _LANES), 0)

        q_row = x_ref[0:1, :]
        q_row = jnp.clip(q_row, -10.0, 10.0)
        xs_row = crow_ref[0:1, :]
        ys_row = crow_ref[1:2, :]
        u_x_row = crow_ref[2:3, :]
        u_y_row = crow_ref[3:4, :]
        ku_x_row = jnp.clip(q_row * u_x_row, -1e6, 1e6)
        ku_y_row = jnp.clip(q_row * u_y_row, -1e6, 1e6)

        xs_col = ccol_ref[0:_M, :]
        ys_col = ccol_ref[_M:2 * _M, :]
        f_obs_col = ccol_ref[2 * _M:3 * _M, 0:1]

        d_ku_dx = _nn_derivative_column(xs_col, xs_row, ku_x_row, jj, ii)
        d_ku_dy = _nn_derivative_column(ys_col, ys_row, ku_y_row, jj, ii)
        f_hat = jnp.clip(d_ku_dx + d_ku_dy, -200.0, 200.0)
        diff = f_obs_col - f_hat
        ii_col = jax.lax.broadcasted_iota(jnp.int32, (_M, 1), 0)
        diff = jnp.where(ii_col < _N, diff, 0.0)
        u_scr[0, 0] = 0.5 * jnp.sum(diff * diff)

    xb = x_ref[...]
    sq = xb * xb
    mrow = jax.lax.broadcasted_iota(jnp.int32, (_COLS, 1), 0)
    mvec = jnp.where(mrow >= _N, 0.5, 0.0).astype(jnp.float32)
    k_val = jax.lax.dot_general(
        sq, mvec,
        dimension_numbers=(((1,), (0,)), ((), ())),
        preferred_element_type=jnp.float32,
    )
    out_ref[...] = jnp.reshape(k_val + u_scr[0, 0], (_BM // 128, 128))


def kernel(x):
    crow, ccol = _consts()
    grid = (_ROWS // _BM,)
    out = pl.pallas_call(
        _body,
        grid=grid,
        in_specs=[
            pl.BlockSpec((8, _LANES), lambda i: (0, 0)),
            pl.BlockSpec((3 * _M, _LANES), lambda i: (0, 0)),
            pl.BlockSpec((_BM, _COLS), lambda i: (i, 0)),
        ],
        out_specs=pl.BlockSpec((_BM // 128, 128), lambda i: (i, 0)),
        out_shape=jax.ShapeDtypeStruct((_ROWS // 128, 128), jnp.float32),
        scratch_shapes=[pltpu.SMEM((1, 1), jnp.float32)],
    )(crow, ccol, x)
    return jnp.reshape(out, (_ROWS, 1))
